# Initial kernel scaffold; baseline (speedup 1.0000x reference)
import jax
import jax.numpy as jnp
from jax.experimental import pallas as pl


def kernel(static_indices, dynamic_indices, obstacle_mask, observability_current, observability_in_memory, previous_visitations, leader_location, follower_location, current_rotations, static_table, dynamic_table):
    raise NotImplementedError("write your pallas kernel here")


# speedup vs baseline: 3.6963x; 3.6963x over previous
import functools

import jax
import jax.numpy as jnp
from jax import lax
from jax.experimental import pallas as pl
from jax.experimental.pallas import tpu as pltpu
from jax.experimental.pallas import tpu_sc as plsc

EDGE = 25
CELLS = EDGE * EDGE
CPAD = 632
NUM_ROT = 6
D = 128
B = 1024
NW = 32
CHUNK = 128
ROWS = B * CPAD
NCHUNK = ROWS // CHUNK
CH_PER_W = NCHUNK // NW
VSCALE = 0.5
NCH = 147


def _sc_gather_fn():
    mesh = plsc.VectorSubcoreMesh(core_axis_name="c", subcore_axis_name="s")

    @functools.partial(
        pl.kernel,
        out_type=jax.ShapeDtypeStruct((ROWS, D), jnp.float32),
        mesh=mesh,
        scratch_types=[
            pltpu.VMEM((CHUNK,), jnp.int32),
            pltpu.VMEM((CHUNK,), jnp.int32),
            pltpu.VMEM((CHUNK, D), jnp.float32),
            pltpu.VMEM((CHUNK, D), jnp.float32),
            pltpu.SemaphoreType.DMA,
            pltpu.SemaphoreType.DMA,
        ],
    )
    def sc_gather(si_hbm, di_hbm, st_hbm, dy_hbm, out_hbm,
                  si_v, di_v, rs_v, rd_v, sem_s, sem_d):
        wid = lax.axis_index("s") * 2 + lax.axis_index("c")

        def body(j, carry):
            ci = wid * CH_PER_W + j
            pltpu.sync_copy(si_hbm.at[ci], si_v)
            pltpu.sync_copy(di_hbm.at[ci], di_v)
            cs = pltpu.async_copy(st_hbm.at[si_v], rs_v, sem_s)
            cd = pltpu.async_copy(dy_hbm.at[di_v], rd_v, sem_d)
            cs.wait()
            cd.wait()

            def add_row(r, c2):
                for v in range(D // 16):
                    sl = pl.ds(v * 16, 16)
                    rs_v[r, sl] = rs_v[r, sl] + rd_v[r, sl]
                return c2

            lax.fori_loop(0, CHUNK, add_row, 0)
            pltpu.sync_copy(rs_v, out_hbm.at[pl.ds(ci * CHUNK, CHUNK)])
            return carry

        lax.fori_loop(0, CH_PER_W, body, 0)

    return sc_gather


def _tc_body(rot_ref, g_ref, obst_ref, ocur_ref, omem_ref, pvis_ref,
             lead_ref, foll_ref, out_ref):
    b = pl.program_id(0)
    rot = rot_ref[b]

    g = g_ref[0]
    i0 = lax.broadcasted_iota(jnp.int32, (D, D), 0)
    i1 = lax.broadcasted_iota(jnp.int32, (D, D), 1)
    ident = jnp.where(i0 == i1, 1.0, 0.0).astype(jnp.float32)
    xt = lax.dot_general(ident, g, (((1,), (1,)), ((), ())),
                         preferred_element_type=jnp.float32)

    m = omem_ref[0]
    out_ref[0, pl.ds(0, D), :] = xt[:, :CELLS] * m
    out_ref[0, pl.ds(D, 1), :] = obst_ref[0] * m
    out_ref[0, pl.ds(D + 1, 1), :] = ocur_ref[0] * m
    out_ref[0, pl.ds(D + 2, 1), :] = m * m

    pv = pvis_ref[0]
    shifted = jnp.zeros((NUM_ROT, CELLS), jnp.float32)
    for s in range(NUM_ROT):
        rolled = pv if s == 0 else jnp.concatenate(
            [pv[NUM_ROT - s:], pv[:NUM_ROT - s]], axis=0)
        shifted = shifted + jnp.where(rot == s, rolled, 0.0)
    out_ref[0, pl.ds(D + 3, NUM_ROT), :] = shifted * (VSCALE * m)
    out_ref[0, pl.ds(D + 9, 1), :] = jnp.sum(pv, axis=0, keepdims=True) * m
    out_ref[0, pl.ds(D + 10, 1), :] = lead_ref[0] * m
    out_ref[0, pl.ds(D + 11, 1), :] = foll_ref[0] * m
    out_ref[0, pl.ds(D + 12, 1), :] = jnp.ones((1, CELLS), jnp.float32)
    rows = lax.broadcasted_iota(jnp.int32, (NUM_ROT, CELLS), 0)
    out_ref[0, pl.ds(D + 13, NUM_ROT), :] = jnp.where(rows == rot, 1.0, 0.0)


def _tc_assemble(rot, g, obst, ocur, omem, pvis, lead, foll):
    return pl.pallas_call(
        _tc_body,
        grid=(B,),
        in_specs=[
            pl.BlockSpec(memory_space=pltpu.SMEM),
            pl.BlockSpec((1, CPAD, D), lambda b: (b, 0, 0)),
            pl.BlockSpec((1, 1, CELLS), lambda b: (b, 0, 0)),
            pl.BlockSpec((1, 1, CELLS), lambda b: (b, 0, 0)),
            pl.BlockSpec((1, 1, CELLS), lambda b: (b, 0, 0)),
            pl.BlockSpec((1, NUM_ROT, CELLS), lambda b: (b, 0, 0)),
            pl.BlockSpec((1, 1, CELLS), lambda b: (b, 0, 0)),
            pl.BlockSpec((1, 1, CELLS), lambda b: (b, 0, 0)),
        ],
        out_specs=pl.BlockSpec((1, NCH, CELLS), lambda b: (b, 0, 0)),
        out_shape=jax.ShapeDtypeStruct((B, NCH, CELLS), jnp.float32),
    )(rot, g, obst, ocur, omem, pvis, lead, foll)


@jax.jit
def kernel(static_indices, dynamic_indices, obstacle_mask,
           observability_current, observability_in_memory,
           previous_visitations, leader_location, follower_location,
           current_rotations, static_table, dynamic_table):
    si = jnp.pad(static_indices.reshape(B, CELLS),
                 ((0, 0), (0, CPAD - CELLS))).reshape(NCHUNK, CHUNK)
    di = jnp.pad(dynamic_indices.reshape(B, CELLS),
                 ((0, 0), (0, CPAD - CELLS))).reshape(NCHUNK, CHUNK)

    g = _sc_gather_fn()(si, di, static_table, dynamic_table)
    g = g.reshape(B, CPAD, D)

    out = _tc_assemble(
        current_rotations, g,
        obstacle_mask.reshape(B, 1, CELLS),
        observability_current.reshape(B, 1, CELLS),
        observability_in_memory.reshape(B, 1, CELLS),
        previous_visitations.reshape(B, NUM_ROT, CELLS),
        leader_location.reshape(B, 1, CELLS),
        follower_location.reshape(B, 1, CELLS),
    )
    return out.reshape(B, NCH, EDGE, EDGE)


# speedup vs baseline: 5.0762x; 1.3733x over previous
import functools

import jax
import jax.numpy as jnp
from jax import lax
from jax.experimental import pallas as pl
from jax.experimental.pallas import tpu as pltpu
from jax.experimental.pallas import tpu_sc as plsc

EDGE = 25
CELLS = EDGE * EDGE
CPAD = 632
NUM_ROT = 6
D = 128
B = 1024
NW = 32
CHUNK = 128
ROWS = B * CPAD
NCHUNK = ROWS // CHUNK
CH_PER_W = NCHUNK // NW
VSCALE = 0.5
NCH = 147


def _sc_gather_fn():
    mesh = plsc.VectorSubcoreMesh(core_axis_name="c", subcore_axis_name="s")
    n = CH_PER_W

    @functools.partial(
        pl.kernel,
        out_type=jax.ShapeDtypeStruct((ROWS, D), jnp.float32),
        mesh=mesh,
        scratch_types=[
            pltpu.VMEM((2, CHUNK), jnp.int32),
            pltpu.VMEM((2, CHUNK), jnp.int32),
            pltpu.VMEM((CHUNK, D), jnp.float32),
            pltpu.VMEM((CHUNK, D), jnp.float32),
            pltpu.VMEM((CHUNK, D), jnp.float32),
            pltpu.VMEM((CHUNK, D), jnp.float32),
            pltpu.SemaphoreType.DMA,
            pltpu.SemaphoreType.DMA,
            pltpu.SemaphoreType.DMA,
            pltpu.SemaphoreType.DMA,
            pltpu.SemaphoreType.DMA,
            pltpu.SemaphoreType.DMA,
        ],
    )
    def sc_gather(si_hbm, di_hbm, st_hbm, dy_hbm, out_hbm,
                  si_v, di_v, rs0, rs1, rd0, rd1,
                  isem0, isem1, gsem0, gsem1, ssem0, ssem1):
        wid = lax.axis_index("s") * 2 + lax.axis_index("c")
        w0 = wid * n
        rs = (rs0, rs1)
        rd = (rd0, rd1)
        isem = (isem0, isem1)
        gsem = (gsem0, gsem1)
        ssem = (ssem0, ssem1)

        def issue_idx(j, b):
            pltpu.async_copy(si_hbm.at[w0 + j], si_v.at[b], isem[b])
            pltpu.async_copy(di_hbm.at[w0 + j], di_v.at[b], isem[b])

        def wait_idx(b):
            pltpu.make_async_copy(si_hbm.at[0], si_v.at[b], isem[b]).wait()
            pltpu.make_async_copy(di_hbm.at[0], di_v.at[b], isem[b]).wait()

        def issue_gathers(b):
            pltpu.async_copy(st_hbm.at[si_v.at[b]], rs[b], gsem[b])
            pltpu.async_copy(dy_hbm.at[di_v.at[b]], rd[b], gsem[b])

        def wait_gathers(b):
            pltpu.make_async_copy(st_hbm.at[pl.ds(0, CHUNK)], rs[b],
                                  gsem[b]).wait()
            pltpu.make_async_copy(st_hbm.at[pl.ds(0, CHUNK)], rd[b],
                                  gsem[b]).wait()

        def issue_store(j, b):
            pltpu.async_copy(rs[b], out_hbm.at[pl.ds((w0 + j) * CHUNK, CHUNK)],
                             ssem[b])

        def wait_store(b):
            pltpu.make_async_copy(rs[b], out_hbm.at[pl.ds(0, CHUNK)],
                                  ssem[b]).wait()

        def add_rows(b):
            def add_row(r, carry):
                for v in range(D // 16):
                    sl = pl.ds(v * 16, 16)
                    rs[b][r, sl] = rs[b][r, sl] + rd[b][r, sl]
                return carry
            lax.fori_loop(0, CHUNK, add_row, 0)

        def stage_c(j, b, first):
            wait_gathers(b)
            if not first:
                issue_idx_cond(j + 2, b)
            else:
                issue_idx(j + 2, b)
            add_rows(b)
            issue_store(j, b)

        def issue_idx_cond(j, b):
            @pl.when(j < n)
            def _():
                issue_idx(j, b)

        def stage_b(j, b):
            wait_idx(b)
            wait_store(b)
            issue_gathers(b)

        issue_idx(0, 0)
        issue_idx(1, 1)
        wait_idx(0)
        issue_gathers(0)
        stage_c(0, 0, True)
        wait_idx(1)
        issue_gathers(1)
        stage_c(1, 1, True)
        stage_b(2, 0)

        def body(i, carry):
            for b in range(2):
                j = 2 * i + b
                stage_c(j, b, False)

                @pl.when(j + 1 < n)
                def _():
                    stage_b(j + 1, 1 - b)
            return carry

        lax.fori_loop(1, n // 2, body, 0)
        wait_store(0)
        wait_store(1)

    return sc_gather


NB = 8


def _tc_body(rot_ref, g_ref, obst_ref, ocur_ref, omem_ref, pvis_ref,
             lead_ref, foll_ref, out_ref):
    b0 = pl.program_id(0) * NB
    i0 = lax.broadcasted_iota(jnp.int32, (D, D), 0)
    i1 = lax.broadcasted_iota(jnp.int32, (D, D), 1)
    ident = jnp.where(i0 == i1, 1.0, 0.0).astype(jnp.float32)

    for nbi in range(NB):
        rot = rot_ref[b0 + nbi]
        g = g_ref[nbi]
        xt = lax.dot_general(ident, g, (((1,), (1,)), ((), ())),
                             preferred_element_type=jnp.float32)

        m = omem_ref[nbi]
        out_ref[nbi, pl.ds(0, D), :] = xt[:, :CELLS] * m
        out_ref[nbi, pl.ds(D, 1), :] = obst_ref[nbi] * m
        out_ref[nbi, pl.ds(D + 1, 1), :] = ocur_ref[nbi] * m
        out_ref[nbi, pl.ds(D + 2, 1), :] = m * m

        pv = pvis_ref[nbi]
        shifted = jnp.zeros((NUM_ROT, CELLS), jnp.float32)
        for s in range(NUM_ROT):
            rolled = pv if s == 0 else jnp.concatenate(
                [pv[NUM_ROT - s:], pv[:NUM_ROT - s]], axis=0)
            shifted = shifted + jnp.where(rot == s, rolled, 0.0)
        out_ref[nbi, pl.ds(D + 3, NUM_ROT), :] = shifted * (VSCALE * m)
        out_ref[nbi, pl.ds(D + 9, 1), :] = (
            jnp.sum(pv, axis=0, keepdims=True) * m)
        out_ref[nbi, pl.ds(D + 10, 1), :] = lead_ref[nbi] * m
        out_ref[nbi, pl.ds(D + 11, 1), :] = foll_ref[nbi] * m
        out_ref[nbi, pl.ds(D + 12, 1), :] = jnp.ones((1, CELLS), jnp.float32)
        rows = lax.broadcasted_iota(jnp.int32, (NUM_ROT, CELLS), 0)
        out_ref[nbi, pl.ds(D + 13, NUM_ROT), :] = jnp.where(rows == rot,
                                                            1.0, 0.0)


def _tc_assemble(rot, g, obst, ocur, omem, pvis, lead, foll):
    return pl.pallas_call(
        _tc_body,
        grid=(B // NB,),
        in_specs=[
            pl.BlockSpec(memory_space=pltpu.SMEM),
            pl.BlockSpec((NB, CPAD, D), lambda b: (b, 0, 0)),
            pl.BlockSpec((NB, 1, CELLS), lambda b: (b, 0, 0)),
            pl.BlockSpec((NB, 1, CELLS), lambda b: (b, 0, 0)),
            pl.BlockSpec((NB, 1, CELLS), lambda b: (b, 0, 0)),
            pl.BlockSpec((NB, NUM_ROT, CELLS), lambda b: (b, 0, 0)),
            pl.BlockSpec((NB, 1, CELLS), lambda b: (b, 0, 0)),
            pl.BlockSpec((NB, 1, CELLS), lambda b: (b, 0, 0)),
        ],
        out_specs=pl.BlockSpec((NB, NCH, CELLS), lambda b: (b, 0, 0)),
        out_shape=jax.ShapeDtypeStruct((B, NCH, CELLS), jnp.float32),
    )(rot, g, obst, ocur, omem, pvis, lead, foll)


@jax.jit
def kernel(static_indices, dynamic_indices, obstacle_mask,
           observability_current, observability_in_memory,
           previous_visitations, leader_location, follower_location,
           current_rotations, static_table, dynamic_table):
    si = jnp.pad(static_indices.reshape(B, CELLS),
                 ((0, 0), (0, CPAD - CELLS))).reshape(NCHUNK, CHUNK)
    di = jnp.pad(dynamic_indices.reshape(B, CELLS),
                 ((0, 0), (0, CPAD - CELLS))).reshape(NCHUNK, CHUNK)

    g = _sc_gather_fn()(si, di, static_table, dynamic_table)
    g = g.reshape(B, CPAD, D)

    out = _tc_assemble(
        current_rotations, g,
        obstacle_mask.reshape(B, 1, CELLS),
        observability_current.reshape(B, 1, CELLS),
        observability_in_memory.reshape(B, 1, CELLS),
        previous_visitations.reshape(B, NUM_ROT, CELLS),
        leader_location.reshape(B, 1, CELLS),
        follower_location.reshape(B, 1, CELLS),
    )
    return out.reshape(B, NCH, EDGE, EDGE)


# speedup vs baseline: 5.0804x; 1.0008x over previous
import functools

import jax
import jax.numpy as jnp
from jax import lax
from jax.experimental import pallas as pl
from jax.experimental.pallas import tpu as pltpu
from jax.experimental.pallas import tpu_sc as plsc

EDGE = 25
CELLS = EDGE * EDGE
CPAD = 632
NUM_ROT = 6
D = 128
B = 1024
NW = 32
CHUNK = 128
ROWS = B * CPAD
NCHUNK = ROWS // CHUNK
CH_PER_W = NCHUNK // NW
VSCALE = 0.5
NCH = 147


def _sc_gather_fn():
    mesh = plsc.VectorSubcoreMesh(core_axis_name="c", subcore_axis_name="s")
    n = CH_PER_W

    NBUF = 3

    @functools.partial(
        pl.kernel,
        out_type=jax.ShapeDtypeStruct((ROWS, D), jnp.float32),
        mesh=mesh,
        scratch_types=[
            pltpu.VMEM((NBUF, CHUNK), jnp.int32),
            pltpu.VMEM((NBUF, CHUNK), jnp.int32),
            pltpu.VMEM((CHUNK, D), jnp.float32),
            pltpu.VMEM((CHUNK, D), jnp.float32),
            pltpu.VMEM((CHUNK, D), jnp.float32),
            pltpu.VMEM((CHUNK, D), jnp.float32),
            pltpu.VMEM((CHUNK, D), jnp.float32),
            pltpu.VMEM((CHUNK, D), jnp.float32),
            pltpu.SemaphoreType.DMA,
            pltpu.SemaphoreType.DMA,
            pltpu.SemaphoreType.DMA,
            pltpu.SemaphoreType.DMA,
            pltpu.SemaphoreType.DMA,
            pltpu.SemaphoreType.DMA,
            pltpu.SemaphoreType.DMA,
            pltpu.SemaphoreType.DMA,
            pltpu.SemaphoreType.DMA,
        ],
    )
    def sc_gather(si_hbm, di_hbm, st_hbm, dy_hbm, out_hbm,
                  si_v, di_v, rs0, rs1, rs2, rd0, rd1, rd2,
                  isem0, isem1, isem2, gsem0, gsem1, gsem2,
                  ssem0, ssem1, ssem2):
        wid = lax.axis_index("s") * 2 + lax.axis_index("c")
        w0 = wid * n
        rs = (rs0, rs1, rs2)
        rd = (rd0, rd1, rd2)
        isem = (isem0, isem1, isem2)
        gsem = (gsem0, gsem1, gsem2)
        ssem = (ssem0, ssem1, ssem2)

        def issue_idx(j, b):
            pltpu.async_copy(si_hbm.at[w0 + j], si_v.at[b], isem[b])
            pltpu.async_copy(di_hbm.at[w0 + j], di_v.at[b], isem[b])

        def wait_idx(b):
            pltpu.make_async_copy(si_hbm.at[0], si_v.at[b], isem[b]).wait()
            pltpu.make_async_copy(di_hbm.at[0], di_v.at[b], isem[b]).wait()

        def issue_gathers(b):
            pltpu.async_copy(st_hbm.at[si_v.at[b]], rs[b], gsem[b])
            pltpu.async_copy(dy_hbm.at[di_v.at[b]], rd[b], gsem[b])

        def wait_gathers(b):
            pltpu.make_async_copy(st_hbm.at[pl.ds(0, CHUNK)], rs[b],
                                  gsem[b]).wait()
            pltpu.make_async_copy(st_hbm.at[pl.ds(0, CHUNK)], rd[b],
                                  gsem[b]).wait()

        def issue_store(j, b):
            pltpu.async_copy(rs[b], out_hbm.at[pl.ds((w0 + j) * CHUNK, CHUNK)],
                             ssem[b])

        def wait_store(b):
            pltpu.make_async_copy(rs[b], out_hbm.at[pl.ds(0, CHUNK)],
                                  ssem[b]).wait()

        def add_rows(b):
            def add_row(r, carry):
                for v in range(D // 16):
                    sl = pl.ds(v * 16, 16)
                    rs[b][r, sl] = rs[b][r, sl] + rd[b][r, sl]
                return carry
            lax.fori_loop(0, CHUNK, add_row, 0)

        def step(j, b, skip_store_wait=False, last_idx=False):
            b2 = (b + 2) % NBUF
            wait_idx(b2)
            if not skip_store_wait:
                wait_store(b2)
            issue_gathers(b2)
            wait_gathers(b)
            if last_idx:
                @pl.when(j + NBUF < n)
                def _():
                    issue_idx(j + NBUF, b)
            else:
                issue_idx(j + NBUF, b)
            add_rows(b)
            issue_store(j, b)

        for j in range(NBUF):
            issue_idx(j, j)
        wait_idx(0)
        issue_gathers(0)
        wait_idx(1)
        issue_gathers(1)
        step(0, 0, skip_store_wait=True)
        step(1, 1)
        step(2, 2)

        def body(i, carry):
            for b in range(NBUF):
                step(NBUF * i + b, b, last_idx=True)
            return carry

        lax.fori_loop(1, (n - 2) // NBUF, body, 0)

        for j, b in ((n - 2, (n - 2) % NBUF), (n - 1, (n - 1) % NBUF)):
            wait_gathers(b)
            add_rows(b)
            issue_store(j, b)
        for b in range(NBUF):
            wait_store(b)

    return sc_gather


NB = 8


def _tc_body(rot_ref, g_ref, obst_ref, ocur_ref, omem_ref, pvis_ref,
             lead_ref, foll_ref, out_ref):
    b0 = pl.program_id(0) * NB
    i0 = lax.broadcasted_iota(jnp.int32, (D, D), 0)
    i1 = lax.broadcasted_iota(jnp.int32, (D, D), 1)
    ident = jnp.where(i0 == i1, 1.0, 0.0).astype(jnp.float32)

    for nbi in range(NB):
        rot = rot_ref[b0 + nbi]
        g = g_ref[nbi]
        xt = lax.dot_general(ident, g, (((1,), (1,)), ((), ())),
                             preferred_element_type=jnp.float32)

        m = omem_ref[nbi]
        out_ref[nbi, pl.ds(0, D), :] = xt[:, :CELLS] * m
        out_ref[nbi, pl.ds(D, 1), :] = obst_ref[nbi] * m
        out_ref[nbi, pl.ds(D + 1, 1), :] = ocur_ref[nbi] * m
        out_ref[nbi, pl.ds(D + 2, 1), :] = m * m

        pv = pvis_ref[nbi]
        shifted = jnp.zeros((NUM_ROT, CELLS), jnp.float32)
        for s in range(NUM_ROT):
            rolled = pv if s == 0 else jnp.concatenate(
                [pv[NUM_ROT - s:], pv[:NUM_ROT - s]], axis=0)
            shifted = shifted + jnp.where(rot == s, rolled, 0.0)
        out_ref[nbi, pl.ds(D + 3, NUM_ROT), :] = shifted * (VSCALE * m)
        out_ref[nbi, pl.ds(D + 9, 1), :] = (
            jnp.sum(pv, axis=0, keepdims=True) * m)
        out_ref[nbi, pl.ds(D + 10, 1), :] = lead_ref[nbi] * m
        out_ref[nbi, pl.ds(D + 11, 1), :] = foll_ref[nbi] * m
        out_ref[nbi, pl.ds(D + 12, 1), :] = jnp.ones((1, CELLS), jnp.float32)
        rows = lax.broadcasted_iota(jnp.int32, (NUM_ROT, CELLS), 0)
        out_ref[nbi, pl.ds(D + 13, NUM_ROT), :] = jnp.where(rows == rot,
                                                            1.0, 0.0)


def _tc_assemble(rot, g, obst, ocur, omem, pvis, lead, foll):
    return pl.pallas_call(
        _tc_body,
        grid=(B // NB,),
        in_specs=[
            pl.BlockSpec(memory_space=pltpu.SMEM),
            pl.BlockSpec((NB, CPAD, D), lambda b: (b, 0, 0)),
            pl.BlockSpec((NB, 1, CELLS), lambda b: (b, 0, 0)),
            pl.BlockSpec((NB, 1, CELLS), lambda b: (b, 0, 0)),
            pl.BlockSpec((NB, 1, CELLS), lambda b: (b, 0, 0)),
            pl.BlockSpec((NB, NUM_ROT, CELLS), lambda b: (b, 0, 0)),
            pl.BlockSpec((NB, 1, CELLS), lambda b: (b, 0, 0)),
            pl.BlockSpec((NB, 1, CELLS), lambda b: (b, 0, 0)),
        ],
        out_specs=pl.BlockSpec((NB, NCH, CELLS), lambda b: (b, 0, 0)),
        out_shape=jax.ShapeDtypeStruct((B, NCH, CELLS), jnp.float32),
    )(rot, g, obst, ocur, omem, pvis, lead, foll)


@jax.jit
def kernel(static_indices, dynamic_indices, obstacle_mask,
           observability_current, observability_in_memory,
           previous_visitations, leader_location, follower_location,
           current_rotations, static_table, dynamic_table):
    si = jnp.pad(static_indices.reshape(B, CELLS),
                 ((0, 0), (0, CPAD - CELLS))).reshape(NCHUNK, CHUNK)
    di = jnp.pad(dynamic_indices.reshape(B, CELLS),
                 ((0, 0), (0, CPAD - CELLS))).reshape(NCHUNK, CHUNK)

    g = _sc_gather_fn()(si, di, static_table, dynamic_table)
    g = g.reshape(B, CPAD, D)

    out = _tc_assemble(
        current_rotations, g,
        obstacle_mask.reshape(B, 1, CELLS),
        observability_current.reshape(B, 1, CELLS),
        observability_in_memory.reshape(B, 1, CELLS),
        previous_visitations.reshape(B, NUM_ROT, CELLS),
        leader_location.reshape(B, 1, CELLS),
        follower_location.reshape(B, 1, CELLS),
    )
    return out.reshape(B, NCH, EDGE, EDGE)
